# baseline (speedup 1.0000x reference)
import functools

import jax
import jax.numpy as jnp
from jax import lax
from jax.experimental import pallas as pl
from jax.experimental.pallas import tpu as pltpu
from jax.experimental.pallas import tpu_sc as plsc

_NEG_T = 0.4
_POS_T = 0.5
_N_PAD = 128
_NW = 32
_CHUNK = 112
_NCHUNK = 12


def _match_kernel(anch_ref, gtr_ref, gtc_ref, idx_ref, tbl_ref, coef_ref,
                  *, inv_h, inv_w):
    b = pl.program_id(0)
    a0 = anch_ref[0:1, :]
    a1 = anch_ref[1:2, :]
    a2 = anch_ref[2:3, :]
    a3 = anch_ref[3:4, :]
    A2x = a0 + a2
    A2y = a1 + a3
    area_a = a2 * a3

    gt_cols = gtc_ref[0]
    X1 = gt_cols[:, 0:1]
    Y1 = gt_cols[:, 1:2]
    GW = gt_cols[:, 2:3]
    GH = gt_cols[:, 3:4]
    C = gt_cols[:, 4:5]
    X2 = X1 + GW
    Y2 = Y1 + GH
    area_g = GW * GH

    ix = jnp.maximum(jnp.minimum(A2x, X2) - jnp.maximum(a0, X1), 0.0)
    iy = jnp.maximum(jnp.minimum(A2y, Y2) - jnp.maximum(a1, Y1), 0.0)
    inter = ix * iy
    union = area_a + area_g - inter
    iou = jnp.where(union > 0.0, inter / jnp.where(union > 0.0, union, 1.0), 0.0)

    mx = jnp.max(iou, axis=0, keepdims=True)
    iota = jax.lax.broadcasted_iota(jnp.int32, iou.shape, 0)
    cand = jnp.where(iou == mx, iota, _N_PAD)
    fidx = jnp.min(cand, axis=0, keepdims=True)

    variant = ((mx < _POS_T).astype(jnp.int32)
               + (mx < _NEG_T).astype(jnp.int32))
    idx_ref[0] = fidx + b * _N_PAD + variant * (8 * _N_PAD)

    zcol = jnp.zeros((_N_PAD, 11), jnp.float32)
    base = jnp.concatenate([Y1, X1, Y2, X2], axis=1)
    tbl_ref[0, 0] = jnp.concatenate([base, C, zcol], axis=1)
    tbl_ref[1, 0] = jnp.concatenate(
        [base, jnp.full((_N_PAD, 1), -2.0, jnp.float32), zcol], axis=1)
    tbl_ref[2, 0] = jnp.concatenate(
        [base, jnp.full((_N_PAD, 1), -1.0, jnp.float32), zcol], axis=1)

    cx0 = (a0 + a2) * 0.5
    cy0 = (a1 + a3) * 0.5
    r0 = 1.0 / (a2 - a0)
    r1 = 1.0 / (a3 - a1)
    zrow = jnp.zeros((3, a0.shape[1]), jnp.float32)
    one = jnp.ones_like(a0)
    coef_ref[0] = jnp.concatenate(
        [cx0 * r0, cy0 * r1, -cx0 * r0, -cy0 * r1, 0.0 * a0, zrow], axis=0)
    coef_ref[1] = jnp.concatenate(
        [-r0 * inv_h, -r1 * inv_w, r0 * inv_h, r1 * inv_w, one, zrow], axis=0)


def _sc_assign(tbl_hbm, idx_hbm, coef_hbm, out_hbm, idx_v, g_v, coef_v, o_v, sem):
    nrows = _CHUNK * _NCHUNK
    wid = lax.axis_index("s") * 2 + lax.axis_index("c")
    aoff = lax.rem(wid, 4) * nrows
    lead = lax.rem(wid, 2) * 64
    aoff_al = pl.multiple_of(aoff - lead, 128)

    pltpu.sync_copy(idx_hbm.at[wid], idx_v)
    copies = [
        pltpu.async_copy(tbl_hbm.at[idx_v.at[k]],
                         g_v.at[pl.ds(k * _CHUNK, _CHUNK)], sem)
        for k in range(_NCHUNK)
    ]
    pltpu.sync_copy(coef_hbm.at[:, :, pl.ds(aoff_al, nrows + 64)], coef_v)
    for c in copies:
        c.wait()

    @pl.loop(0, nrows // 16)
    def _(j):
        row0 = j * 16
        riota = lax.iota(jnp.int32, 16) + row0
        crow = row0 + lead
        for c in range(5):
            cidx = jnp.full((16,), c, jnp.int32)
            g = plsc.load_gather(g_v, [riota, cidx])
            a = coef_v[0, c, pl.ds(crow, 16)]
            bb = coef_v[1, c, pl.ds(crow, 16)]
            o_v[c, pl.ds(row0, 16)] = a + bb * g

    pltpu.sync_copy(o_v, out_hbm.at[wid])


def kernel(images, gt_boxes, gt_classes, anchor_boxes):
    B, N = gt_boxes.shape[0], gt_boxes.shape[1]
    M = anchor_boxes.shape[0]
    H, W = images.shape[1], images.shape[2]
    BM = B * M
    nrows = _CHUNK * _NCHUNK

    anch_t = anchor_boxes.T
    gt5 = jnp.concatenate([gt_boxes, gt_classes], axis=-1)
    gt_cols = jnp.pad(gt5, ((0, 0), (0, _N_PAD - N), (0, 3)))

    body = functools.partial(_match_kernel, inv_h=1.0 / H, inv_w=1.0 / W)
    gidx, tbl, coef = pl.pallas_call(
        body,
        grid=(B,),
        in_specs=[
            pl.BlockSpec((4, M), lambda b: (0, 0)),
            pl.BlockSpec((1, 8, _N_PAD), lambda b: (b, 0, 0)),
            pl.BlockSpec((1, _N_PAD, 8), lambda b: (b, 0, 0)),
        ],
        out_specs=[
            pl.BlockSpec((1, 1, M), lambda b: (b, 0, 0)),
            pl.BlockSpec((3, 1, _N_PAD, 16), lambda b: (0, b, 0, 0)),
            pl.BlockSpec((2, 8, M), lambda b: (0, 0, 0)),
        ],
        out_shape=[
            jax.ShapeDtypeStruct((B, 1, M), jnp.int32),
            jax.ShapeDtypeStruct((3, B, _N_PAD, 16), jnp.float32),
            jax.ShapeDtypeStruct((2, 8, M), jnp.float32),
        ],
    )(anch_t, jnp.transpose(gt_cols, (0, 2, 1)), gt_cols)

    idx3 = gidx.reshape(_NW, _NCHUNK, _CHUNK)
    tbl2 = tbl.reshape(3 * B * _N_PAD, 16)

    mesh = plsc.VectorSubcoreMesh(core_axis_name="c", subcore_axis_name="s")
    sc = functools.partial(
        pl.kernel, mesh=mesh,
        compiler_params=pltpu.CompilerParams(needs_layout_passes=False,
                                             use_tc_tiling_on_sc=False),
        out_type=jax.ShapeDtypeStruct((_NW, 5, nrows), jnp.float32),
        scratch_types=[
            pltpu.VMEM((_NCHUNK, _CHUNK), jnp.int32),
            pltpu.VMEM((nrows, 16), jnp.float32),
            pltpu.VMEM((2, 8, nrows + 64), jnp.float32),
            pltpu.VMEM((5, nrows), jnp.float32),
            pltpu.SemaphoreType.DMA,
        ],
    )(_sc_assign)
    out = sc(tbl2, idx3, coef)

    outg = jnp.transpose(out, (0, 2, 1)).reshape(B, M, 5)
    return outg[..., :4], outg[..., 4]


# speedup vs baseline: 1.3073x; 1.3073x over previous
import functools

import jax
import jax.numpy as jnp
from jax import lax
from jax.experimental import pallas as pl
from jax.experimental.pallas import tpu as pltpu
from jax.experimental.pallas import tpu_sc as plsc

_NEG_T = 0.4
_POS_T = 0.5
_N_PAD = 128
_NW = 32
_CHUNK = 112
_NCHUNK = 12


def _match_kernel(anch_ref, gtr_ref, gtc_ref, idx_ref, tbl_ref, coef_ref,
                  *, inv_h, inv_w):
    b = pl.program_id(0)
    a0 = anch_ref[0:1, :]
    a1 = anch_ref[1:2, :]
    a2 = anch_ref[2:3, :]
    a3 = anch_ref[3:4, :]
    A2x = a0 + a2
    A2y = a1 + a3
    area_a = a2 * a3

    gt_cols = gtc_ref[0]
    X1 = gt_cols[:, 0:1]
    Y1 = gt_cols[:, 1:2]
    GW = gt_cols[:, 2:3]
    GH = gt_cols[:, 3:4]
    C = gt_cols[:, 4:5]
    X2 = X1 + GW
    Y2 = Y1 + GH
    area_g = GW * GH

    ix = jnp.maximum(jnp.minimum(A2x, X2) - jnp.maximum(a0, X1), 0.0)
    iy = jnp.maximum(jnp.minimum(A2y, Y2) - jnp.maximum(a1, Y1), 0.0)
    inter = ix * iy
    union = area_a + area_g - inter
    iou = jnp.where(union > 0.0, inter / jnp.where(union > 0.0, union, 1.0), 0.0)

    mx = jnp.max(iou, axis=0, keepdims=True)
    iota = jax.lax.broadcasted_iota(jnp.int32, iou.shape, 0)
    cand = jnp.where(iou == mx, iota, _N_PAD)
    fidx = jnp.min(cand, axis=0, keepdims=True)

    variant = ((mx < _POS_T).astype(jnp.int32)
               + (mx < _NEG_T).astype(jnp.int32))
    idx_ref[0] = fidx + variant * _N_PAD

    zcol = jnp.zeros((_N_PAD, 11), jnp.float32)
    base = jnp.concatenate([Y1, X1, Y2, X2], axis=1)
    tbl_ref[0, 0:_N_PAD] = jnp.concatenate([base, C, zcol], axis=1)
    tbl_ref[0, _N_PAD:2 * _N_PAD] = jnp.concatenate(
        [base, jnp.full((_N_PAD, 1), -2.0, jnp.float32), zcol], axis=1)
    tbl_ref[0, 2 * _N_PAD:3 * _N_PAD] = jnp.concatenate(
        [base, jnp.full((_N_PAD, 1), -1.0, jnp.float32), zcol], axis=1)

    cx0 = (a0 + a2) * 0.5
    cy0 = (a1 + a3) * 0.5
    r0 = 1.0 / (a2 - a0)
    r1 = 1.0 / (a3 - a1)
    zrow = jnp.zeros((3, a0.shape[1]), jnp.float32)
    one = jnp.ones_like(a0)
    coef_ref[0] = jnp.concatenate(
        [cx0 * r0, cy0 * r1, -cx0 * r0, -cy0 * r1, 0.0 * a0, zrow], axis=0)
    coef_ref[1] = jnp.concatenate(
        [-r0 * inv_h, -r1 * inv_w, r0 * inv_h, r1 * inv_w, one, zrow], axis=0)


def _sc_assign(tbl_hbm, idx_hbm, coef_hbm, out_hbm, idx_v, tbl_v, coef_v, o_v,
               sem, sem2, sem3):
    nrows = _CHUNK * _NCHUNK
    wid = lax.axis_index("s") * 2 + lax.axis_index("c")
    aoff = lax.rem(wid, 4) * nrows
    lead = lax.rem(wid, 2) * 64
    aoff_al = pl.multiple_of(aoff - lead, 128)

    c1 = pltpu.async_copy(idx_hbm.at[wid], idx_v, sem)
    c2 = pltpu.async_copy(tbl_hbm.at[lax.div(wid, 4)], tbl_v, sem2)
    c3 = pltpu.async_copy(coef_hbm.at[:, :, pl.ds(aoff_al, nrows + 64)],
                          coef_v, sem3)
    c1.wait()
    c2.wait()
    c3.wait()

    @pl.loop(0, nrows // 16)
    def _(j):
        row0 = j * 16
        idx16 = idx_v[j]
        crow = row0 + lead
        for c in range(5):
            cidx = jnp.full((16,), c, jnp.int32)
            g = plsc.load_gather(tbl_v, [idx16, cidx])
            a = coef_v[0, c, pl.ds(crow, 16)]
            bb = coef_v[1, c, pl.ds(crow, 16)]
            o_v[c, pl.ds(row0, 16)] = a + bb * g

    pltpu.sync_copy(o_v, out_hbm.at[wid])


def kernel(images, gt_boxes, gt_classes, anchor_boxes):
    B, N = gt_boxes.shape[0], gt_boxes.shape[1]
    M = anchor_boxes.shape[0]
    H, W = images.shape[1], images.shape[2]
    BM = B * M
    nrows = _CHUNK * _NCHUNK

    anch_t = anchor_boxes.T
    gt5 = jnp.concatenate([gt_boxes, gt_classes], axis=-1)
    gt_cols = jnp.pad(gt5, ((0, 0), (0, _N_PAD - N), (0, 3)))

    body = functools.partial(_match_kernel, inv_h=1.0 / H, inv_w=1.0 / W)
    gidx, tbl, coef = pl.pallas_call(
        body,
        grid=(B,),
        in_specs=[
            pl.BlockSpec((4, M), lambda b: (0, 0)),
            pl.BlockSpec((1, 8, _N_PAD), lambda b: (b, 0, 0)),
            pl.BlockSpec((1, _N_PAD, 8), lambda b: (b, 0, 0)),
        ],
        out_specs=[
            pl.BlockSpec((1, 1, M), lambda b: (b, 0, 0)),
            pl.BlockSpec((1, 3 * _N_PAD, 16), lambda b: (b, 0, 0)),
            pl.BlockSpec((2, 8, M), lambda b: (0, 0, 0)),
        ],
        out_shape=[
            jax.ShapeDtypeStruct((B, 1, M), jnp.int32),
            jax.ShapeDtypeStruct((B, 3 * _N_PAD, 16), jnp.float32),
            jax.ShapeDtypeStruct((2, 8, M), jnp.float32),
        ],
    )(anch_t, jnp.transpose(gt_cols, (0, 2, 1)), gt_cols)

    idx3 = gidx.reshape(_NW, nrows // 16, 16)

    mesh = plsc.VectorSubcoreMesh(core_axis_name="c", subcore_axis_name="s")
    sc = functools.partial(
        pl.kernel, mesh=mesh,
        compiler_params=pltpu.CompilerParams(needs_layout_passes=False,
                                             use_tc_tiling_on_sc=False),
        out_type=jax.ShapeDtypeStruct((_NW, 5, nrows), jnp.float32),
        scratch_types=[
            pltpu.VMEM((nrows // 16, 16), jnp.int32),
            pltpu.VMEM((3 * _N_PAD, 16), jnp.float32),
            pltpu.VMEM((2, 8, nrows + 64), jnp.float32),
            pltpu.VMEM((5, nrows), jnp.float32),
            pltpu.SemaphoreType.DMA,
            pltpu.SemaphoreType.DMA,
            pltpu.SemaphoreType.DMA,
        ],
    )(_sc_assign)
    out = sc(tbl, idx3, coef)

    outg = jnp.transpose(out, (0, 2, 1)).reshape(B, M, 5)
    return outg[..., :4], outg[..., 4]
